# Initial kernel scaffold; baseline (speedup 1.0000x reference)
import jax
import jax.numpy as jnp
from jax.experimental import pallas as pl


def kernel(lut, x):
    raise NotImplementedError("write your pallas kernel here")


# speedup vs baseline: 1405.4435x; 1405.4435x over previous
import functools

import jax
import jax.numpy as jnp
from jax import lax
from jax.experimental import pallas as pl
from jax.experimental.pallas import tpu as pltpu
from jax.experimental.pallas import tpu_sc as plsc

_DIM = 33
_TBL = _DIM ** 3
_BINSIZE = 1.000001 / (_DIM - 1)
_NW = 32
_NPIX = 8 * 512 * 512
_PER_W = _NPIX // _NW
_IMG = 512 * 512
_CHUNK = 4096
_L = 16


@functools.partial(
    pl.kernel,
    out_type=jax.ShapeDtypeStruct((24, _IMG), jnp.float32),
    mesh=plsc.VectorSubcoreMesh(core_axis_name="c", subcore_axis_name="s"),
    compiler_params=pltpu.CompilerParams(needs_layout_passes=False),
    scratch_types=[
        pltpu.VMEM((3 * _TBL,), jnp.float32),
        pltpu.VMEM((_CHUNK,), jnp.float32),
        pltpu.VMEM((_CHUNK,), jnp.float32),
        pltpu.VMEM((_CHUNK,), jnp.float32),
    ],
)
def _lut3d_sc(lut_hbm, x_hbm, out_hbm, lut_v, rv, gv, bv):
    wid = lax.axis_index("s") * 2 + lax.axis_index("c")
    img = wid // 4
    base_px = (wid % 4) * _PER_W

    pltpu.sync_copy(lut_hbm, lut_v)

    def chunk_body(ci, carry):
        off = base_px + ci * _CHUNK
        pltpu.sync_copy(x_hbm.at[3 * img + 0, pl.ds(off, _CHUNK)], rv)
        pltpu.sync_copy(x_hbm.at[3 * img + 1, pl.ds(off, _CHUNK)], gv)
        pltpu.sync_copy(x_hbm.at[3 * img + 2, pl.ds(off, _CHUNK)], bv)

        def vec_body(vi, c2):
            s = vi * _L
            r = rv[pl.ds(s, _L)]
            g = gv[pl.ds(s, _L)]
            b = bv[pl.ds(s, _L)]
            rs = r / _BINSIZE
            gs = g / _BINSIZE
            bs = b / _BINSIZE
            ri = rs.astype(jnp.int32)
            gi = gs.astype(jnp.int32)
            bi = bs.astype(jnp.int32)
            rd = rs - ri.astype(jnp.float32)
            gd = gs - gi.astype(jnp.float32)
            bd = bs - bi.astype(jnp.float32)
            base = ri + gi * _DIM + bi * (_DIM * _DIM)
            outs = []
            for c in range(3):
                bc = base + c * _TBL
                v000 = plsc.load_gather(lut_v, [bc])
                v100 = plsc.load_gather(lut_v, [bc + 1])
                v010 = plsc.load_gather(lut_v, [bc + _DIM])
                v110 = plsc.load_gather(lut_v, [bc + (_DIM + 1)])
                v001 = plsc.load_gather(lut_v, [bc + _DIM * _DIM])
                v101 = plsc.load_gather(lut_v, [bc + (_DIM * _DIM + 1)])
                v011 = plsc.load_gather(lut_v, [bc + (_DIM * _DIM + _DIM)])
                v111 = plsc.load_gather(lut_v, [bc + (_DIM * _DIM + _DIM + 1)])
                v00 = v000 + rd * (v100 - v000)
                v10 = v010 + rd * (v110 - v010)
                v01 = v001 + rd * (v101 - v001)
                v11 = v011 + rd * (v111 - v011)
                v0 = v00 + gd * (v10 - v00)
                v1 = v01 + gd * (v11 - v01)
                outs.append(v0 + bd * (v1 - v0))
            rv[pl.ds(s, _L)] = outs[0]
            gv[pl.ds(s, _L)] = outs[1]
            bv[pl.ds(s, _L)] = outs[2]
            return c2

        lax.fori_loop(0, _CHUNK // _L, vec_body, 0)
        pltpu.sync_copy(rv, out_hbm.at[3 * img + 0, pl.ds(off, _CHUNK)])
        pltpu.sync_copy(gv, out_hbm.at[3 * img + 1, pl.ds(off, _CHUNK)])
        pltpu.sync_copy(bv, out_hbm.at[3 * img + 2, pl.ds(off, _CHUNK)])
        return carry

    lax.fori_loop(0, _PER_W // _CHUNK, chunk_body, 0)


def kernel(lut, x):
    lut_flat = lut.reshape(3 * _TBL)
    x_flat = x.reshape(24, _IMG)
    out = _lut3d_sc(lut_flat, x_flat)
    return out.reshape(8, 3, 512, 512)
